# baseline (speedup 1.0000x reference)
import functools

import jax
import jax.numpy as jnp
from jax import lax
from jax.experimental import pallas as pl
from jax.experimental.pallas import tpu as pltpu
from jax.experimental.pallas import tpu_sc as plsc

NC, NS = 2, 16
NW = NC * NS
V = 48
D = 64
L = 16
P = 256
NB = 2


@functools.lru_cache(maxsize=None)
def _make_sc_gather(B: int):
    B2 = B // 2
    assert B2 % (NW * P * NB) == 0
    p_per_w = B2 // NW
    n_chunks = p_per_w // P
    mesh = plsc.VectorSubcoreMesh(core_axis_name="c", subcore_axis_name="s")

    @functools.partial(
        pl.kernel,
        mesh=mesh,
        out_type=jax.ShapeDtypeStruct((B2, 2 * D), jnp.float32),
        scratch_types=(
            [pltpu.VMEM_SHARED((V * V, 2 * D), jnp.float32)]
            + [pltpu.VMEM((2 * P,), jnp.int32) for _ in range(NB)]
            + [pltpu.VMEM((P,), jnp.int32) for _ in range(NB)]
            + [pltpu.VMEM((P, 2 * D), jnp.float32) for _ in range(NB)]
            + [pltpu.SemaphoreType.DMA((NB,)),
               pltpu.SemaphoreType.DMA((NB,)),
               pltpu.SemaphoreType.DMA((NB,))]
        ),
        compiler_params=pltpu.CompilerParams(use_tc_tiling_on_sc=True,
                                             needs_layout_passes=False),
    )
    def k(table2_hbm, idx_hbm, out_hbm, table2_s, *bufs):
        idx_v = bufs[0:NB]
        cidx_v = bufs[NB:2 * NB]
        rows_v = bufs[2 * NB:3 * NB]
        idx_sem, gat_sem, out_sem = bufs[3 * NB:3 * NB + 3]
        wid = lax.axis_index("s") * NC + lax.axis_index("c")
        i0 = wid * 2 * p_per_w
        o0 = wid * p_per_w

        def idx_copy(g, s):
            return pltpu.make_async_copy(
                idx_hbm.at[pl.ds(i0 + g * 2 * P, 2 * P)],
                idx_v[s], idx_sem.at[s])

        def gat_copy(s, j):
            return pltpu.make_async_copy(
                table2_s.at[cidx_v[s].at[pl.ds(j * 128, 128)]],
                rows_v[s].at[pl.ds(j * 128, 128)], gat_sem.at[s])

        def out_copy(g, s):
            return pltpu.make_async_copy(
                rows_v[s],
                out_hbm.at[pl.ds(o0 + g * P, P)], out_sem.at[s])

        @pl.when(lax.axis_index("s") == 0)
        def _():
            pltpu.sync_copy(table2_hbm, table2_s)

        plsc.subcore_barrier()
        for s in range(NB):
            idx_copy(s, s).start()

        def combine(s):
            src = idx_v[s]
            dst = cidx_v[s]
            iota2 = lax.iota(jnp.int32, L) * 2
            for q in range(P // L):
                ev = plsc.load_gather(src, [iota2 + (q * 2 * L)])
                od = plsc.load_gather(src, [iota2 + (q * 2 * L + 1)])
                dst[pl.ds(q * L, L)] = ev * V + od

        def outer(i, carry):
            g0 = i * NB
            for s in range(NB):
                g = g0 + s
                idx_copy(g, s).wait()

                @pl.when(g >= NB)
                def _():
                    out_copy(g - NB, s).wait()

                combine(s)
                for j in range(P // 128):
                    gat_copy(s, j).start()
                for j in range(P // 128):
                    gat_copy(s, j).wait()

                @pl.when(g + NB < n_chunks)
                def _():
                    idx_copy(g + NB, s).start()

                out_copy(g, s).start()
            return carry

        lax.fori_loop(0, n_chunks // NB, outer, 0)
        for s in range(NB):
            out_copy(n_chunks - NB + s, s).wait()

    return k


def kernel(timedelta, table):
    Bt, H = timedelta.shape
    B = Bt * H
    idx = timedelta.reshape(B).astype(jnp.int32)
    table2 = jnp.concatenate(
        [
            jnp.broadcast_to(table[:, None, :], (V, V, D)),
            jnp.broadcast_to(table[None, :, :], (V, V, D)),
        ],
        axis=-1,
    ).reshape(V * V, 2 * D)
    out = _make_sc_gather(B)(table2, idx)
    return out.reshape(Bt, H, D)


# speedup vs baseline: 1.7950x; 1.7950x over previous
import functools

import jax
import jax.numpy as jnp
from jax import lax
from jax.experimental import pallas as pl
from jax.experimental.pallas import tpu as pltpu
from jax.experimental.pallas import tpu_sc as plsc

NC, NS = 2, 16
NW = NC * NS
V = 48
D = 64
DP = 128
P = 256
NB = 2


@functools.lru_cache(maxsize=None)
def _make_sc_gather(Bt: int, H: int):
    B = Bt * H
    assert B % (NW * P * NB) == 0
    b_per_w = B // NW
    n_chunks = b_per_w // P
    mesh = plsc.VectorSubcoreMesh(core_axis_name="c", subcore_axis_name="s")

    @functools.partial(
        pl.kernel,
        mesh=mesh,
        out_type=jax.ShapeDtypeStruct((B, D), jnp.float32),
        scratch_types=(
            [pltpu.VMEM_SHARED((V, D), jnp.float32)]
            + [pltpu.VMEM((P,), jnp.int32) for _ in range(NB)]
            + [pltpu.VMEM((P, D), jnp.float32) for _ in range(NB)]
            + [pltpu.SemaphoreType.DMA((NB,)),
               pltpu.SemaphoreType.DMA((NB,)),
               pltpu.SemaphoreType.DMA((NB,))]
        ),
        compiler_params=pltpu.CompilerParams(use_tc_tiling_on_sc=True,
                                             needs_layout_passes=False),
    )
    def k(table_hbm, idx_hbm, out_hbm, table_s, *bufs):
        idx_v = bufs[0:NB]
        rows_v = bufs[NB:2 * NB]
        idx_sem, gat_sem, out_sem = bufs[2 * NB:2 * NB + 3]
        out_flat = out_hbm
        wid = lax.axis_index("s") * NC + lax.axis_index("c")
        row0 = wid * b_per_w

        def idx_copy(g, s):
            return pltpu.make_async_copy(
                idx_hbm.at[pl.ds(row0 + g * P, P)],
                idx_v[s], idx_sem.at[s])

        def gat_copy(s, j):
            return pltpu.make_async_copy(
                table_s.at[idx_v[s].at[pl.ds(j * 128, 128)]],
                rows_v[s].at[pl.ds(j * 128, 128)], gat_sem.at[s])

        def out_copy(g, s):
            return pltpu.make_async_copy(
                rows_v[s],
                out_flat.at[pl.ds(row0 + g * P, P)], out_sem.at[s])

        @pl.when(lax.axis_index("s") == 0)
        def _():
            pltpu.sync_copy(table_hbm, table_s)

        plsc.subcore_barrier()
        for s in range(NB):
            idx_copy(s, s).start()

        def outer(i, carry):
            g0 = i * NB
            for s in range(NB):
                g = g0 + s
                idx_copy(g, s).wait()

                @pl.when(g >= NB)
                def _():
                    out_copy(g - NB, s).wait()

                for j in range(P // 128):
                    gat_copy(s, j).start()
                for j in range(P // 128):
                    gat_copy(s, j).wait()

                @pl.when(g + NB < n_chunks)
                def _():
                    idx_copy(g + NB, s).start()

                out_copy(g, s).start()
            return carry

        lax.fori_loop(0, n_chunks // NB, outer, 0)
        for s in range(NB):
            out_copy(n_chunks - NB + s, s).wait()

    return k


def kernel(timedelta, table):
    Bt, H = timedelta.shape
    idx = timedelta.reshape(Bt * H).astype(jnp.int32)
    return _make_sc_gather(Bt, H)(table, idx).reshape(Bt, H, D)
